# baseline (speedup 1.0000x reference)
import functools
import math

import jax
import jax.numpy as jnp
from jax import lax
from jax.experimental import pallas as pl
from jax.experimental.pallas import tpu as pltpu
from jax.experimental.pallas import tpu_sc as plsc

_TEMP = 0.05
_N = 100000
_D = 64
_B = 1024
_BLK = 2048
_GRID = (_N + _BLK - 1) // _BLK
_LN2 = math.log(2.0)
_SHIFT = 115.0

_NC = 2
_NS = 16
_L = 16
_NW = _NC * _NS
_DPW = _D // _NW


def _lse_body(x_ref, f_ref, out_ref, m_ref, s_ref):
    j = pl.program_id(0)
    x = x_ref[...]

    @pl.when(j == 0)
    def _init():
        norm = jnp.sqrt(jnp.sum(x * x, axis=1, keepdims=True))
        m_ref[...] = norm - _SHIFT
        s_ref[...] = jnp.zeros_like(s_ref[...])

    f = f_ref[...]
    logits2 = lax.dot_general(
        x.astype(jnp.bfloat16), f.astype(jnp.bfloat16),
        (((1,), (0,)), ((), ())),
        preferred_element_type=jnp.float32)
    m2 = m_ref[...]

    @pl.when(j < _GRID - 1)
    def _mid():
        y = jnp.exp2(logits2 - m2).astype(jnp.bfloat16)
        s_ref[...] += jnp.sum(y, axis=1, keepdims=True).astype(jnp.float32)

    @pl.when(j == _GRID - 1)
    def _last():
        col = j * _BLK + lax.broadcasted_iota(jnp.int32, (1, _BLK), 1)
        masked = jnp.where(col < _N, logits2, -1e30)
        s = s_ref[...] + jnp.sum(jnp.exp2(masked - m2), axis=1, keepdims=True)
        logz = m2 * _LN2 + jnp.log(s)
        out_ref[0, 0] = jnp.sum(logz) * (1.0 / _B)


_lse = pl.pallas_call(
    _lse_body,
    grid=(_GRID,),
    in_specs=[
        pl.BlockSpec((_B, _D), lambda j: (0, 0)),
        pl.BlockSpec((_D, _BLK), lambda j: (0, j)),
    ],
    out_specs=pl.BlockSpec((1, 1), lambda j: (0, 0), memory_space=pltpu.SMEM),
    out_shape=jax.ShapeDtypeStruct((1, 1), jnp.float32),
    scratch_shapes=[
        pltpu.VMEM((_B, 1), jnp.float32),
        pltpu.VMEM((_B, 1), jnp.float32),
    ],
)


def _tgt_body(xt_hbm, t_hbm, ft_hbm, out_hbm, t_v, row_v, xrow_v, acc_v, sem):
    wid = lax.axis_index("s") * _NC + lax.axis_index("c")
    pltpu.sync_copy(t_hbm, t_v)
    acc = jnp.zeros((_L,), jnp.float32)
    for k in range(_DPW):
        c = wid * _DPW + k
        cp = pltpu.async_copy(ft_hbm.at[c], row_v, sem)
        pltpu.sync_copy(xt_hbm.at[c], xrow_v)
        cp.wait()
        for g in range(_B // _L):
            tg = t_v[pl.ds(g * _L, _L)]
            fv = plsc.load_gather(row_v, [tg])
            acc = acc + fv * xrow_v[pl.ds(g * _L, _L)]
    acc_v[...] = acc
    pltpu.sync_copy(acc_v, out_hbm.at[wid])


@functools.cache
def _tgt():
    return pl.kernel(
        _tgt_body,
        mesh=plsc.VectorSubcoreMesh(core_axis_name="c", subcore_axis_name="s"),
        out_type=jax.ShapeDtypeStruct((_NW, _L), jnp.float32),
        scratch_types=[
            pltpu.VMEM((_B,), jnp.int32),
            pltpu.VMEM((_N,), jnp.float32),
            pltpu.VMEM((_B,), jnp.float32),
            pltpu.VMEM((_L,), jnp.float32),
            pltpu.SemaphoreType.DMA,
        ],
        compiler_params=pltpu.CompilerParams(needs_layout_passes=False),
    )


def kernel(inputs, targets, features):
    x2 = inputs * (1.0 / (_TEMP * _LN2))
    ft = features.T
    mean_logz = _lse(x2, ft)[0, 0]
    partials = _tgt()(x2.T, targets, ft)
    return mean_logz - jnp.sum(partials) * (_LN2 / _B)


# speedup vs baseline: 1.1380x; 1.1380x over previous
import functools
import math

import jax
import jax.numpy as jnp
from jax import lax
from jax.experimental import pallas as pl
from jax.experimental.pallas import tpu as pltpu
from jax.experimental.pallas import tpu_sc as plsc

_TEMP = 0.05
_N = 100000
_D = 64
_B = 1024
_BLK = 4096
_GRID = (_N + _BLK - 1) // _BLK
_LN2 = math.log(2.0)
_SHIFT = 115.0

_NC = 2
_NS = 16
_L = 16
_NW = _NC * _NS
_DPW = _D // _NW


def _lse_body(x_ref, f_ref, out_ref, m_ref, s_ref):
    j = pl.program_id(0)
    x = x_ref[...]

    @pl.when(j == 0)
    def _init():
        norm = jnp.sqrt(jnp.sum(x * x, axis=1, keepdims=True))
        m_ref[...] = norm - _SHIFT
        s_ref[...] = jnp.zeros_like(s_ref[...])

    f = f_ref[...]
    logits2 = lax.dot_general(
        x.astype(jnp.bfloat16), f.astype(jnp.bfloat16),
        (((1,), (0,)), ((), ())),
        preferred_element_type=jnp.float32)
    m2 = m_ref[...]

    @pl.when(j < _GRID - 1)
    def _mid():
        s_ref[...] += jnp.sum(jnp.exp2(logits2 - m2), axis=1, keepdims=True)

    @pl.when(j == _GRID - 1)
    def _last():
        col = j * _BLK + lax.broadcasted_iota(jnp.int32, (1, _BLK), 1)
        masked = jnp.where(col < _N, logits2, -1e30)
        s = s_ref[...] + jnp.sum(jnp.exp2(masked - m2), axis=1, keepdims=True)
        logz = m2 * _LN2 + jnp.log(s)
        out_ref[0, 0] = jnp.sum(logz) * (1.0 / _B)


_lse = pl.pallas_call(
    _lse_body,
    grid=(_GRID,),
    in_specs=[
        pl.BlockSpec((_B, _D), lambda j: (0, 0)),
        pl.BlockSpec((_D, _BLK), lambda j: (0, j)),
    ],
    out_specs=pl.BlockSpec((1, 1), lambda j: (0, 0), memory_space=pltpu.SMEM),
    out_shape=jax.ShapeDtypeStruct((1, 1), jnp.float32),
    scratch_shapes=[
        pltpu.VMEM((_B, 1), jnp.float32),
        pltpu.VMEM((_B, 1), jnp.float32),
    ],
)


def _tgt_body(xt_hbm, t_hbm, ft_hbm, out_hbm, t_v, row_v, xrow_v, acc_v, sem):
    wid = lax.axis_index("s") * _NC + lax.axis_index("c")
    pltpu.sync_copy(t_hbm, t_v)
    acc = jnp.zeros((_L,), jnp.float32)
    for k in range(_DPW):
        c = wid * _DPW + k
        cp = pltpu.async_copy(ft_hbm.at[c], row_v, sem)
        pltpu.sync_copy(xt_hbm.at[c], xrow_v)
        cp.wait()
        for g in range(_B // _L):
            tg = t_v[pl.ds(g * _L, _L)]
            fv = plsc.load_gather(row_v, [tg])
            acc = acc + fv * xrow_v[pl.ds(g * _L, _L)]
    acc_v[...] = acc
    pltpu.sync_copy(acc_v, out_hbm.at[wid])


@functools.cache
def _tgt():
    return pl.kernel(
        _tgt_body,
        mesh=plsc.VectorSubcoreMesh(core_axis_name="c", subcore_axis_name="s"),
        out_type=jax.ShapeDtypeStruct((_NW, _L), jnp.float32),
        scratch_types=[
            pltpu.VMEM((_B,), jnp.int32),
            pltpu.VMEM((_N,), jnp.float32),
            pltpu.VMEM((_B,), jnp.float32),
            pltpu.VMEM((_L,), jnp.float32),
            pltpu.SemaphoreType.DMA,
        ],
        compiler_params=pltpu.CompilerParams(needs_layout_passes=False),
    )


def kernel(inputs, targets, features):
    x2 = inputs * (1.0 / (_TEMP * _LN2))
    ft = features.T
    mean_logz = _lse(x2, ft)[0, 0]
    partials = _tgt()(x2.T, targets, ft)
    return mean_logz - jnp.sum(partials) * (_LN2 / _B)


# speedup vs baseline: 1.1567x; 1.0165x over previous
import functools
import math

import jax
import jax.numpy as jnp
from jax import lax
from jax.experimental import pallas as pl
from jax.experimental.pallas import tpu as pltpu
from jax.experimental.pallas import tpu_sc as plsc

_TEMP = 0.05
_N = 100000
_D = 64
_B = 1024
_BLK = 8192
_GRID = (_N + _BLK - 1) // _BLK
_LN2 = math.log(2.0)
_SHIFT = 115.0

_NC = 2
_NS = 16
_L = 16
_NW = _NC * _NS
_DPW = _D // _NW


def _lse_body(x_ref, f_ref, out_ref, m_ref, s_ref):
    j = pl.program_id(0)
    x = x_ref[...]

    @pl.when(j == 0)
    def _init():
        norm = jnp.sqrt(jnp.sum(x * x, axis=1, keepdims=True))
        m_ref[...] = norm - _SHIFT
        s_ref[...] = jnp.zeros_like(s_ref[...])

    f = f_ref[...]
    logits2 = lax.dot_general(
        x.astype(jnp.bfloat16), f.astype(jnp.bfloat16),
        (((1,), (0,)), ((), ())),
        preferred_element_type=jnp.float32)
    m2 = m_ref[...]

    @pl.when(j < _GRID - 1)
    def _mid():
        s_ref[...] += jnp.sum(jnp.exp2(logits2 - m2), axis=1, keepdims=True)

    @pl.when(j == _GRID - 1)
    def _last():
        col = j * _BLK + lax.broadcasted_iota(jnp.int32, (1, _BLK), 1)
        masked = jnp.where(col < _N, logits2, -1e30)
        s = s_ref[...] + jnp.sum(jnp.exp2(masked - m2), axis=1, keepdims=True)
        logz = m2 * _LN2 + jnp.log(s)
        out_ref[0, 0] = jnp.sum(logz) * (1.0 / _B)


_lse = pl.pallas_call(
    _lse_body,
    grid=(_GRID,),
    in_specs=[
        pl.BlockSpec((_B, _D), lambda j: (0, 0)),
        pl.BlockSpec((_D, _BLK), lambda j: (0, j)),
    ],
    out_specs=pl.BlockSpec((1, 1), lambda j: (0, 0), memory_space=pltpu.SMEM),
    out_shape=jax.ShapeDtypeStruct((1, 1), jnp.float32),
    scratch_shapes=[
        pltpu.VMEM((_B, 1), jnp.float32),
        pltpu.VMEM((_B, 1), jnp.float32),
    ],
)


def _tgt_body(xt_hbm, t_hbm, ft_hbm, out_hbm, t_v, row_v, xrow_v, acc_v, sem):
    wid = lax.axis_index("s") * _NC + lax.axis_index("c")
    pltpu.sync_copy(t_hbm, t_v)
    acc = jnp.zeros((_L,), jnp.float32)
    for k in range(_DPW):
        c = wid * _DPW + k
        cp = pltpu.async_copy(ft_hbm.at[c], row_v, sem)
        pltpu.sync_copy(xt_hbm.at[c], xrow_v)
        cp.wait()
        for g in range(_B // _L):
            tg = t_v[pl.ds(g * _L, _L)]
            fv = plsc.load_gather(row_v, [tg])
            acc = acc + fv * xrow_v[pl.ds(g * _L, _L)]
    acc_v[...] = acc
    pltpu.sync_copy(acc_v, out_hbm.at[wid])


@functools.cache
def _tgt():
    return pl.kernel(
        _tgt_body,
        mesh=plsc.VectorSubcoreMesh(core_axis_name="c", subcore_axis_name="s"),
        out_type=jax.ShapeDtypeStruct((_NW, _L), jnp.float32),
        scratch_types=[
            pltpu.VMEM((_B,), jnp.int32),
            pltpu.VMEM((_N,), jnp.float32),
            pltpu.VMEM((_B,), jnp.float32),
            pltpu.VMEM((_L,), jnp.float32),
            pltpu.SemaphoreType.DMA,
        ],
        compiler_params=pltpu.CompilerParams(needs_layout_passes=False),
    )


def kernel(inputs, targets, features):
    x2 = inputs * (1.0 / (_TEMP * _LN2))
    ft = features.T
    mean_logz = _lse(x2, ft)[0, 0]
    partials = _tgt()(x2.T, targets, ft)
    return mean_logz - jnp.sum(partials) * (_LN2 / _B)
